# baseline (speedup 1.0000x reference)
import functools

import jax
import jax.numpy as jnp
from jax.experimental import pallas as pl
from jax.experimental.pallas import tpu as pltpu
from jax.experimental.pallas import tpu_sc as plsc

B = 16384
D = 64
LANES = 128
NC = 2
NS = 16
NW = NC * NS
RPW = B // NW
CHUNK = 128
NCHUNK = RPW // CHUNK
HALF = RPW // 2


def _sc_gather_body(idx2_u, idx2_m, users2d, movies2d,
                    lmf_uw, lmf_mw, mlp_uw, mlp_mw, b_u, b_m,
                    gu_out, gm_out, gmu_out, gmm_out, gub_out, gmb_out,
                    iu, im, uv, mv, buf_a, buf_b, bias_buf, sem_a, sem_b,
                    sem_c):
    wid = jax.lax.axis_index("s") * NC + jax.lax.axis_index("c")
    base = wid * RPW

    pltpu.sync_copy(idx2_u.at[pl.ds(wid * NCHUNK, NCHUNK)], iu)
    pltpu.sync_copy(idx2_m.at[pl.ds(wid * NCHUNK, NCHUNK)], im)
    pltpu.sync_copy(users2d.at[pl.ds(wid * NCHUNK, NCHUNK)], uv)
    pltpu.sync_copy(movies2d.at[pl.ds(wid * NCHUNK, NCHUNK)], mv)

    bias_cps = [
        pltpu.async_copy(b_u.at[uv.at[j]], bias_buf.at[j], sem_c)
        for j in range(NCHUNK)
    ] + [
        pltpu.async_copy(b_m.at[mv.at[j]], bias_buf.at[NCHUNK + j], sem_c)
        for j in range(NCHUNK)
    ]

    stages = []
    for table, idx, out in ((lmf_uw, iu, gu_out), (mlp_uw, iu, gmu_out),
                            (lmf_mw, im, gm_out), (mlp_mw, im, gmm_out)):
        for h in range(2):
            stages.append((table, idx, out, h))

    def fire(stage, buf, sem):
        table, idx, _, h = stage
        return [
            pltpu.async_copy(table.at[idx.at[h * 2 + j]],
                             buf.at[pl.ds(j * CHUNK, CHUNK)], sem)
            for j in range(2)
        ]

    def drain_write(stage, buf, cps):
        _, _, out, h = stage
        for c in cps:
            c.wait()
        pltpu.sync_copy(buf, out.at[pl.ds(base + h * HALF, HALF)])

    bufs = (buf_a, buf_b)
    sems = (sem_a, sem_b)
    cps = [None, None]
    cps[0] = fire(stages[0], buf_a, sem_a)
    cps[1] = fire(stages[1], buf_b, sem_b)
    for s in range(len(stages)):
        slot = s % 2
        drain_write(stages[s], bufs[slot], cps[slot])
        if s + 2 < len(stages):
            cps[slot] = fire(stages[s + 2], bufs[slot], sems[slot])

    for c in bias_cps:
        c.wait()
    pltpu.sync_copy(bias_buf.at[pl.ds(0, NCHUNK)],
                    gub_out.at[pl.ds(wid * NCHUNK, NCHUNK)])
    pltpu.sync_copy(bias_buf.at[pl.ds(NCHUNK, NCHUNK)],
                    gmb_out.at[pl.ds(wid * NCHUNK, NCHUNK)])


_sc_gather = functools.partial(
    pl.kernel,
    out_type=[
        jax.ShapeDtypeStruct((B, LANES), jnp.float32),
        jax.ShapeDtypeStruct((B, LANES), jnp.float32),
        jax.ShapeDtypeStruct((B, LANES), jnp.float32),
        jax.ShapeDtypeStruct((B, LANES), jnp.float32),
        jax.ShapeDtypeStruct((B // LANES, LANES), jnp.float32),
        jax.ShapeDtypeStruct((B // LANES, LANES), jnp.float32),
    ],
    mesh=plsc.VectorSubcoreMesh(
        core_axis_name="c", subcore_axis_name="s", num_cores=NC,
        num_subcores=NS),
    compiler_params=pltpu.CompilerParams(use_tc_tiling_on_sc=True),
    scratch_types=[
        pltpu.VMEM((NCHUNK, CHUNK), jnp.int32),
        pltpu.VMEM((NCHUNK, CHUNK), jnp.int32),
        pltpu.VMEM((NCHUNK, CHUNK), jnp.int32),
        pltpu.VMEM((NCHUNK, CHUNK), jnp.int32),
        pltpu.VMEM((HALF, LANES), jnp.float32),
        pltpu.VMEM((HALF, LANES), jnp.float32),
        pltpu.VMEM((2 * NCHUNK, CHUNK), jnp.float32),
        pltpu.SemaphoreType.DMA,
        pltpu.SemaphoreType.DMA,
        pltpu.SemaphoreType.DMA,
    ],
)(_sc_gather_body)


RB = 2048


def _tc_dense_body(users, movies, gu, gm, gmu, gmm, ub, mb,
                   W1, b1, W2, b2, W3, b3, Wf, bf, out):
    pu = (users[...] & 1) == 1
    pm = (movies[...] & 1) == 1

    def half(g, p):
        return jnp.where(p, g[:, D:], g[:, :D])

    uw = half(gu[...], pu)
    mw = half(gm[...], pm)
    mlp_u = half(gmu[...], pu)
    mlp_m = half(gmm[...], pm)

    lmf = jax.nn.sigmoid(jnp.sum(uw * mw, axis=1, keepdims=True)
                         + ub[...] + mb[...])
    h = jnp.dot(mlp_u, W1[0:D, :], preferred_element_type=jnp.float32)
    h += jnp.dot(mlp_m, W1[D:2 * D, :], preferred_element_type=jnp.float32)
    h = jax.nn.relu(h + b1[...])
    h = jax.nn.relu(jnp.dot(h, W2[...], preferred_element_type=jnp.float32)
                    + b2[...])
    mlp = jax.nn.sigmoid(
        jnp.dot(h, W3[...], preferred_element_type=jnp.float32) + b3[...])
    x = jax.nn.sigmoid(lmf * Wf[0, 0] + mlp * Wf[1, 0] + bf[0, 0])
    out[...] = x * 4.5 + 0.5


def _tc_dense(users, movies, gu, gm, gmu, gmm, ub, mb,
              W1, b1, W2, b2, W3, b3, Wf, bf):
    row = lambda i: (i, 0)
    rep = lambda i: (0, 0)
    return pl.pallas_call(
        _tc_dense_body,
        grid=(B // RB,),
        in_specs=[
            pl.BlockSpec((RB, 1), row),
            pl.BlockSpec((RB, 1), row),
            pl.BlockSpec((RB, LANES), row),
            pl.BlockSpec((RB, LANES), row),
            pl.BlockSpec((RB, LANES), row),
            pl.BlockSpec((RB, LANES), row),
            pl.BlockSpec((RB, 1), row),
            pl.BlockSpec((RB, 1), row),
            pl.BlockSpec((2 * D, D), rep),
            pl.BlockSpec((1, D), rep),
            pl.BlockSpec((D, 16), rep),
            pl.BlockSpec((1, 16), rep),
            pl.BlockSpec((16, 1), rep),
            pl.BlockSpec((1, 1), rep),
            pl.BlockSpec((2, 1), rep),
            pl.BlockSpec((1, 1), rep),
        ],
        out_specs=pl.BlockSpec((RB, 1), row),
        out_shape=jax.ShapeDtypeStruct((B, 1), jnp.float32),
    )(users, movies, gu, gm, gmu, gmm, ub, mb,
      W1, b1, W2, b2, W3, b3, Wf, bf)


def kernel(users, movies, lmf_user_w, lmf_user_b, lmf_movie_w, lmf_movie_b,
           mlp_user_w, mlp_movie_w, W1, b1, W2, b2, W3, b3, Wf, bf):
    users = users.astype(jnp.int32)
    movies = movies.astype(jnp.int32)
    grid2 = (B // CHUNK, CHUNK)
    idx2_u = (users >> 1).reshape(grid2)
    idx2_m = (movies >> 1).reshape(grid2)

    NU = 1000000
    NM = 100000
    gu, gm, gmu, gmm, gub, gmb = _sc_gather(
        idx2_u, idx2_m, users.reshape(grid2), movies.reshape(grid2),
        lmf_user_w.reshape(NU // 2, 2 * D),
        lmf_movie_w.reshape(NM // 2, 2 * D),
        mlp_user_w.reshape(NU // 2, 2 * D),
        mlp_movie_w.reshape(NM // 2, 2 * D),
        lmf_user_b.T.reshape(-1), lmf_movie_b.T.reshape(-1))

    return _tc_dense(
        users.reshape(B, 1), movies.reshape(B, 1),
        gu, gm, gmu, gmm, gub.reshape(B, 1), gmb.reshape(B, 1),
        W1, b1.reshape(1, D), W2, b2.reshape(1, 16), W3, b3.reshape(1, 1),
        Wf, bf.reshape(1, 1))


# speedup vs baseline: 1.4348x; 1.4348x over previous
import functools

import jax
import jax.numpy as jnp
from jax.experimental import pallas as pl
from jax.experimental.pallas import tpu as pltpu
from jax.experimental.pallas import tpu_sc as plsc

B = 16384
D = 64
LANES = 128
NC = 2
NS = 16
NW = NC * NS
RPW = B // NW
CHUNK = 128
NCHUNK = RPW // CHUNK
HALF = RPW // 2


def _sc_gather_body(idx_u, idx_m, lmf_uw, lmf_mw, mlp_uw, mlp_mw, b_u, b_m,
                    gu_out, gm_out, gmu_out, gmm_out, gub_out, gmb_out,
                    iu, im, buf_a, buf_b, bias_buf, sem_a, sem_b, sem_c):
    wid = jax.lax.axis_index("s") * NC + jax.lax.axis_index("c")
    base = wid * RPW

    pltpu.sync_copy(idx_u.at[pl.ds(wid * NCHUNK, NCHUNK)], iu)
    pltpu.sync_copy(idx_m.at[pl.ds(wid * NCHUNK, NCHUNK)], im)

    bias_cps = [
        pltpu.async_copy(b_u.at[iu.at[j]], bias_buf.at[j], sem_c)
        for j in range(NCHUNK)
    ] + [
        pltpu.async_copy(b_m.at[im.at[j]], bias_buf.at[NCHUNK + j], sem_c)
        for j in range(NCHUNK)
    ]

    stages = []
    for table, idx, out in ((lmf_uw, iu, gu_out), (mlp_uw, iu, gmu_out),
                            (lmf_mw, im, gm_out), (mlp_mw, im, gmm_out)):
        for h in range(2):
            stages.append((table, idx, out, h))

    def fire(stage, buf, sem):
        table, idx, _, h = stage
        for j in range(2 * h, 2 * h + 2):
            for g in range(CHUNK // 16):
                vec = idx[j, pl.ds(g * 16, 16)]
                for t in range(16):
                    k = (j - 2 * h) * CHUNK + g * 16 + t
                    pltpu.async_copy(table.at[pl.ds(vec[t], 1)],
                                     buf.at[pl.ds(k, 1)], sem)

    def drain_write(stage, buf, sem):
        table, _, out, h = stage
        pltpu.make_async_copy(table.at[pl.ds(0, HALF)], buf, sem).wait()
        pltpu.sync_copy(buf, out.at[pl.ds(base + h * HALF, HALF)])

    bufs = (buf_a, buf_b)
    sems = (sem_a, sem_b)
    fire(stages[0], bufs[0], sems[0])
    fire(stages[1], bufs[1], sems[1])
    for s in range(8):
        slot = s % 2
        drain_write(stages[s], bufs[slot], sems[slot])
        if s + 2 < 8:
            fire(stages[s + 2], bufs[slot], sems[slot])

    for c in bias_cps:
        c.wait()
    pltpu.sync_copy(bias_buf.at[pl.ds(0, NCHUNK)],
                    gub_out.at[pl.ds(wid * NCHUNK, NCHUNK)])
    pltpu.sync_copy(bias_buf.at[pl.ds(NCHUNK, NCHUNK)],
                    gmb_out.at[pl.ds(wid * NCHUNK, NCHUNK)])


_sc_gather = functools.partial(
    pl.kernel,
    out_type=[
        jax.ShapeDtypeStruct((B, D), jnp.float32),
        jax.ShapeDtypeStruct((B, D), jnp.float32),
        jax.ShapeDtypeStruct((B, D), jnp.float32),
        jax.ShapeDtypeStruct((B, D), jnp.float32),
        jax.ShapeDtypeStruct((B // LANES, LANES), jnp.float32),
        jax.ShapeDtypeStruct((B // LANES, LANES), jnp.float32),
    ],
    mesh=plsc.VectorSubcoreMesh(
        core_axis_name="c", subcore_axis_name="s", num_cores=NC,
        num_subcores=NS),
    compiler_params=pltpu.CompilerParams(use_tc_tiling_on_sc=True),
    scratch_types=[
        pltpu.VMEM((NCHUNK, CHUNK), jnp.int32),
        pltpu.VMEM((NCHUNK, CHUNK), jnp.int32),
        pltpu.VMEM((HALF, D), jnp.float32),
        pltpu.VMEM((HALF, D), jnp.float32),
        pltpu.VMEM((2 * NCHUNK, CHUNK), jnp.float32),
        pltpu.SemaphoreType.DMA,
        pltpu.SemaphoreType.DMA,
        pltpu.SemaphoreType.DMA,
    ],
)(_sc_gather_body)


RB = 2048


def _tc_dense_body(gu, gm, gmu, gmm, ub, mb,
                   W1, b1, W2, b2, W3, b3, Wf, bf, out):
    lmf = jax.nn.sigmoid(jnp.sum(gu[...] * gm[...], axis=1, keepdims=True)
                         + ub[...] + mb[...])
    h = jnp.dot(gmu[...], W1[0:D, :], preferred_element_type=jnp.float32)
    h += jnp.dot(gmm[...], W1[D:2 * D, :], preferred_element_type=jnp.float32)
    h = jax.nn.relu(h + b1[...])
    h = jax.nn.relu(jnp.dot(h, W2[...], preferred_element_type=jnp.float32)
                    + b2[...])
    mlp = jax.nn.sigmoid(
        jnp.dot(h, W3[...], preferred_element_type=jnp.float32) + b3[...])
    x = jax.nn.sigmoid(lmf * Wf[0, 0] + mlp * Wf[1, 0] + bf[0, 0])
    out[...] = x * 4.5 + 0.5


def _tc_dense(gu, gm, gmu, gmm, ub, mb,
              W1, b1, W2, b2, W3, b3, Wf, bf):
    row = lambda i: (i, 0)
    rep = lambda i: (0, 0)
    return pl.pallas_call(
        _tc_dense_body,
        grid=(B // RB,),
        in_specs=[
            pl.BlockSpec((RB, D), row),
            pl.BlockSpec((RB, D), row),
            pl.BlockSpec((RB, D), row),
            pl.BlockSpec((RB, D), row),
            pl.BlockSpec((RB, 1), row),
            pl.BlockSpec((RB, 1), row),
            pl.BlockSpec((2 * D, D), rep),
            pl.BlockSpec((1, D), rep),
            pl.BlockSpec((D, 16), rep),
            pl.BlockSpec((1, 16), rep),
            pl.BlockSpec((16, 1), rep),
            pl.BlockSpec((1, 1), rep),
            pl.BlockSpec((2, 1), rep),
            pl.BlockSpec((1, 1), rep),
        ],
        out_specs=pl.BlockSpec((RB, 1), row),
        out_shape=jax.ShapeDtypeStruct((B, 1), jnp.float32),
    )(gu, gm, gmu, gmm, ub, mb,
      W1, b1, W2, b2, W3, b3, Wf, bf)


def kernel(users, movies, lmf_user_w, lmf_user_b, lmf_movie_w, lmf_movie_b,
           mlp_user_w, mlp_movie_w, W1, b1, W2, b2, W3, b3, Wf, bf):
    users = users.astype(jnp.int32)
    movies = movies.astype(jnp.int32)
    grid2 = (B // CHUNK, CHUNK)

    gu, gm, gmu, gmm, gub, gmb = _sc_gather(
        users.reshape(grid2), movies.reshape(grid2),
        lmf_user_w, lmf_movie_w, mlp_user_w, mlp_movie_w,
        lmf_user_b.T.reshape(-1), lmf_movie_b.T.reshape(-1))

    return _tc_dense(
        gu, gm, gmu, gmm, gub.reshape(B, 1), gmb.reshape(B, 1),
        W1, b1.reshape(1, D), W2, b2.reshape(1, 16), W3, b3.reshape(1, 1),
        Wf, bf.reshape(1, 1))
